# Initial kernel scaffold; baseline (speedup 1.0000x reference)
import jax
import jax.numpy as jnp
from jax.experimental import pallas as pl


def kernel(x, x_64, lstm_params):
    raise NotImplementedError("write your pallas kernel here")


# speedup vs baseline: 7.2732x; 7.2732x over previous
import functools

import jax
import jax.numpy as jnp
from jax.experimental import pallas as pl
from jax.experimental.pallas import tpu as pltpu

_B, _N = 16, 8192
_NBINS = 64
_INS = [64, 512, 256, 128, 128, 256]
_HIDS = [256, 128, 64, 64, 128, 256]
_NEG = float("-inf")
_NPB = 1024
_CHUNK = 1024


def _segmax_kernel(x_ref, f_ref, seg_ref, cnt_ref):
    bi = pl.program_id(1)

    @pl.when(bi == 0)
    def _init():
        seg_ref[...] = jnp.full(seg_ref.shape, _NEG, jnp.float32)
        cnt_ref[...] = jnp.zeros(cnt_ref.shape, jnp.float32)
        seg_ref[0, 3] = jnp.zeros(seg_ref.shape[2:], jnp.float32)

    nchunks = _N // _CHUNK
    for a in range(3):
        c = x_ref[0, a : a + 1, :]
        mn = jnp.min(c, axis=1, keepdims=True)
        ids = jnp.clip(jnp.floor((c - mn) * 64.0), 0.0, 63.0)
        idcs = [ids[:, ch * _CHUNK : (ch + 1) * _CHUNK] for ch in range(nchunks)]

        cnt = cnt_ref[0, a]
        binio = jax.lax.broadcasted_iota(
            jnp.int32, (_NBINS, _CHUNK), 0).astype(jnp.float32)
        ones_row = jnp.ones((1, _CHUNK), jnp.float32)
        for ch in range(nchunks):
            oh = (idcs[ch] == binio).astype(jnp.float32)
            cnt = cnt + jax.lax.dot_general(
                ones_row, oh, (((1,), (1,)), ((), ())),
                preferred_element_type=jnp.float32)
        cnt_ref[0, a] = cnt

        def bin_body(j, _):
            jf = j.astype(jnp.float32)
            m128 = None
            for ch in range(nchunks):
                fc = f_ref[0, :, ch * _CHUNK : (ch + 1) * _CHUNK]
                mk = idcs[ch] == jf
                fm = jnp.where(mk, fc, _NEG)
                t = jnp.maximum(fm[:, :512], fm[:, 512:])
                t = jnp.maximum(t[:, :256], t[:, 256:])
                t = jnp.maximum(t[:, :128], t[:, 128:])
                m128 = t if m128 is None else jnp.maximum(m128, t)
            m = jnp.max(m128, axis=1)
            prev = seg_ref[0, a, pl.ds(j, 1), :, :]
            seg_ref[0, a, pl.ds(j, 1), :, :] = jnp.maximum(
                prev, m.reshape(1, 1, 64))
            return 0

        jax.lax.fori_loop(0, _NBINS, bin_body, 0)


def _run_segmax(x, x_64, interpret=False):
    grid = (2, _B // 2)
    seg, cnt = pl.pallas_call(
        _segmax_kernel,
        grid=grid,
        in_specs=[
            pl.BlockSpec((1, 3, _N), lambda h, b: (h * (_B // 2) + b, 0, 0)),
            pl.BlockSpec((1, 64, _N), lambda h, b: (h * (_B // 2) + b, 0, 0)),
        ],
        out_specs=[
            pl.BlockSpec((1, 4, _NBINS, 1, 64), lambda h, b: (h, 0, 0, 0, 0)),
            pl.BlockSpec((1, 4, 1, 64), lambda h, b: (h, 0, 0, 0)),
        ],
        out_shape=[
            jax.ShapeDtypeStruct((2, 4, _NBINS, 1, 64), jnp.float32),
            jax.ShapeDtypeStruct((2, 4, 1, 64), jnp.float32),
        ],
        compiler_params=pltpu.CompilerParams(
            dimension_semantics=("parallel", "arbitrary"),
            vmem_limit_bytes=50 * 1024 * 1024,
        ),
        interpret=interpret,
    )(x, x_64)
    return seg, cnt


def _lstm_layer(seq, w_ref, u_ref, b_ref, s, xwf, xwb, hf_s, hb_s, H):
    four_h = 4 * H
    dn = (((1,), (1,)), ((), ()))
    xw_f = jax.lax.dot_general(seq, w_ref[s, 0], dn,
                               preferred_element_type=jnp.float32) + b_ref[s, 0]
    xw_b = jax.lax.dot_general(seq, w_ref[s, 1], dn,
                               preferred_element_type=jnp.float32) + b_ref[s, 1]
    xwf[...] = xw_f.reshape(64, 1, four_h)
    xwb[...] = xw_b.reshape(64, 1, four_h)

    def gates(g, c):
        i = jax.nn.sigmoid(g[:, :H])
        f = jax.nn.sigmoid(g[:, H : 2 * H])
        gg = jnp.tanh(g[:, 2 * H : 3 * H])
        o = jax.nn.sigmoid(g[:, 3 * H : 4 * H])
        c2 = f * c + i * gg
        return o * jnp.tanh(c2), c2

    def body(t, carry):
        hf, cf, hb, cb = carry
        gf = xwf[pl.ds(t, 1), 0, :] + jax.lax.dot_general(
            hf, u_ref[s, 0], dn, preferred_element_type=jnp.float32)
        gb = xwb[pl.ds(63 - t, 1), 0, :] + jax.lax.dot_general(
            hb, u_ref[s, 1], dn, preferred_element_type=jnp.float32)
        hf2, cf2 = gates(gf, cf)
        hb2, cb2 = gates(gb, cb)
        hf_s[pl.ds(t, 1), 0, :] = hf2
        hb_s[pl.ds(63 - t, 1), 0, :] = hb2
        return hf2, cf2, hb2, cb2

    z = jnp.zeros((1, H), jnp.float32)
    jax.lax.fori_loop(0, 64, body, (z, z, z, z))
    return jnp.concatenate([hf_s[:, 0, :], hb_s[:, 0, :]], axis=1)


def _lstm_kernel(*refs):
    seg_ref = refs[0]
    w_refs = refs[1:7]
    u_refs = refs[7:13]
    b_refs = refs[13:19]
    out_ref = refs[19]
    scr = refs[20:]
    xw_by_h = {256: (scr[0], scr[1]), 128: (scr[2], scr[3]), 64: (scr[4], scr[5])}
    hs_by_h = {256: (scr[6], scr[7]), 128: (scr[8], scr[9]), 64: (scr[10], scr[11])}

    for s in range(2):
        seq = jnp.maximum(seg_ref[0, s, :, 0, :], seg_ref[1, s, :, 0, :])
        for l in range(6):
            H = _HIDS[l]
            xwf, xwb = xw_by_h[H]
            hf_s, hb_s = hs_by_h[H]
            seq = _lstm_layer(seq, w_refs[l], u_refs[l], b_refs[l], s,
                              xwf, xwb, hf_s, hb_s, H)
        out_ref[s * 64 : (s + 1) * 64, :] = seq


def _run_lstm(seg, w_l, u_l, b_l, interpret=False):
    in_specs = [pl.BlockSpec((2, 2, _NBINS, 1, 64), lambda h: (0, h, 0, 0, 0))]
    for l in range(6):
        I, H = _INS[l], _HIDS[l]
        in_specs.append(pl.BlockSpec((2, 2, 4 * H, I), lambda h: (h, 0, 0, 0)))
    for l in range(6):
        H = _HIDS[l]
        in_specs.append(pl.BlockSpec((2, 2, 4 * H, H), lambda h: (h, 0, 0, 0)))
    for l in range(6):
        H = _HIDS[l]
        in_specs.append(pl.BlockSpec((2, 2, 1, 4 * H), lambda h: (h, 0, 0, 0)))

    scratch = []
    for fh in (1024, 512, 256):
        scratch += [pltpu.VMEM((64, 1, fh), jnp.float32)] * 2
    for hh in (256, 128, 64):
        scratch += [pltpu.VMEM((64, 1, hh), jnp.float32)] * 2

    h_all = pl.pallas_call(
        _lstm_kernel,
        grid=(2,),
        in_specs=in_specs,
        out_specs=pl.BlockSpec((128, 512), lambda h: (h, 0)),
        out_shape=jax.ShapeDtypeStruct((256, 512), jnp.float32),
        scratch_shapes=scratch,
        compiler_params=pltpu.CompilerParams(
            dimension_semantics=("parallel",),
            vmem_limit_bytes=56 * 1024 * 1024,
        ),
        interpret=interpret,
    )(seg, *w_l, *u_l, *b_l)
    return h_all


def _unpool_kernel(h_ref, cnt_ref, out_ref):
    b = pl.program_id(0)
    nb = pl.program_id(1)
    base = (b * _N + nb * _NPB).astype(jnp.float32)
    k = base + jax.lax.broadcasted_iota(
        jnp.int32, (1, _NPB), 1).astype(jnp.float32)

    lt = (jax.lax.broadcasted_iota(jnp.int32, (_NBINS, _NBINS), 1)
          <= jax.lax.broadcasted_iota(jnp.int32, (_NBINS, _NBINS), 0)
          ).astype(jnp.float32)
    ohs = []
    for a in range(3):
        ca = cnt_ref[0, a, :, :] + cnt_ref[1, a, :, :]
        csum_col = jax.lax.dot_general(
            lt, ca, (((1,), (1,)), ((), ())),
            preferred_element_type=jnp.float32)
        le = (csum_col <= k).astype(jnp.float32)
        leprev = jnp.concatenate(
            [jnp.ones((1, _NPB), jnp.float32), le[:63, :]], axis=0)
        ohs.append(leprev - le)
    ohs.append(jnp.zeros((_NBINS, _NPB), jnp.float32))
    oh = jnp.concatenate(ohs, axis=0)

    out_ref[0] = jax.lax.dot_general(
        h_ref[...], oh, (((0,), (0,)), ((), ())),
        preferred_element_type=jnp.float32)


def _run_unpool(h_all, cnt, interpret=False):
    return pl.pallas_call(
        _unpool_kernel,
        grid=(_B, _N // _NPB),
        in_specs=[
            pl.BlockSpec((256, 512), lambda b, nb: (0, 0)),
            pl.BlockSpec((2, 4, 1, 64), lambda b, nb: (0, 0, 0, 0)),
        ],
        out_specs=pl.BlockSpec((1, 512, _NPB), lambda b, nb: (b, 0, nb)),
        out_shape=jax.ShapeDtypeStruct((_B, 512, _N), jnp.float32),
        compiler_params=pltpu.CompilerParams(
            dimension_semantics=("parallel", "arbitrary"),
            vmem_limit_bytes=50 * 1024 * 1024,
        ),
        interpret=interpret,
    )(h_all, cnt)


def _pack_params(lstm_params):
    w_l, u_l, b_l = [], [], []
    for l in range(6):
        I, H = _INS[l], _HIDS[l]
        wax, uax, bax = [], [], []
        for a in range(3):
            Wf, Uf, bf, cf, Wb, Ub, bb, cb = lstm_params[a][l]
            wax.append(jnp.stack([Wf, Wb]))
            uax.append(jnp.stack([Uf, Ub]))
            bax.append(jnp.stack([(bf + cf).reshape(1, 4 * H),
                                  (bb + cb).reshape(1, 4 * H)]))
        wax.append(jnp.zeros((2, 4 * H, I), jnp.float32))
        uax.append(jnp.zeros((2, 4 * H, H), jnp.float32))
        bax.append(jnp.zeros((2, 1, 4 * H), jnp.float32))
        w_l.append(jnp.stack(wax))
        u_l.append(jnp.stack(uax))
        b_l.append(jnp.stack(bax))
    return w_l, u_l, b_l


def _kernel_impl(x, x_64, lstm_params, interpret=False):
    w_l, u_l, b_l = _pack_params(lstm_params)
    seg, cnt = _run_segmax(x, x_64, interpret=interpret)
    h_all = _run_lstm(seg, w_l, u_l, b_l, interpret=interpret)
    return _run_unpool(h_all, cnt, interpret=interpret)


def kernel(x, x_64, lstm_params):
    return _kernel_impl(x, x_64, lstm_params, interpret=False)
